# baseline (speedup 1.0000x reference)
import functools

import jax
import jax.numpy as jnp
from jax import lax
from jax.experimental import pallas as pl
from jax.experimental.pallas import tpu as pltpu
from jax.experimental.pallas import tpu_sc as plsc

N_NODES = 1000000
EMBED_DIM = 64
BATCH = 16384

_INFO = plsc.get_sparse_core_info()
_NC = _INFO.num_cores
_NS = _INFO.num_subcores
_NW = _NC * _NS
_ROWS_PER_W = BATCH // _NW
_LANES = 16
_DGROUPS = EMBED_DIM // _LANES
_NBLOCKS = _ROWS_PER_W // _LANES


def _sc_kernel(head_hbm, tail_hbm, table_hbm, rel_hbm, out_hbm,
               hidx_v, tidx_v, hrows_v, trows_v, rel_v, out_v, sem):
    wid = lax.axis_index("s") * _NC + lax.axis_index("c")
    base = wid * _ROWS_PER_W

    pltpu.sync_copy(rel_hbm, rel_v)
    pltpu.sync_copy(head_hbm.at[pl.ds(base, _ROWS_PER_W)], hidx_v)
    pltpu.sync_copy(tail_hbm.at[pl.ds(base, _ROWS_PER_W)], tidx_v)

    def fire_group(g, carry):
        row0 = g * _LANES
        hv = hidx_v[pl.ds(row0, _LANES)]
        tv = tidx_v[pl.ds(row0, _LANES)]
        quad0 = g * (_LANES // 4)
        for i in range(_LANES):
            dst_row = quad0 + i // 4
            dst_col = (i % 4) * (EMBED_DIM // 2)
            pltpu.async_copy(
                table_hbm.at[hv[i]],
                hrows_v.at[dst_row, pl.ds(dst_col, EMBED_DIM // 2)], sem)
            pltpu.async_copy(
                table_hbm.at[tv[i]],
                trows_v.at[dst_row, pl.ds(dst_col, EMBED_DIM // 2)], sem)
        return carry

    lax.fori_loop(0, _NBLOCKS, fire_group, 0)

    pltpu.make_async_copy(table_hbm.at[pl.ds(0, _ROWS_PER_W)], hrows_v,
                          sem).wait()
    pltpu.make_async_copy(table_hbm.at[pl.ds(0, _ROWS_PER_W)], trows_v,
                          sem).wait()

    rel_regs = [rel_v[pl.ds(j * _LANES, _LANES)] for j in range(_DGROUPS)]

    lane_iota = lax.iota(jnp.int32, _LANES)
    onehot = [(lane_iota == i).astype(jnp.float32) for i in range(_LANES)]

    def row_groups(rows_ref, rq, col):
        lo = rows_ref[rq, pl.ds(col, _LANES)]
        hi = rows_ref[rq, pl.ds(col + _LANES, _LANES)]
        return (plsc.bitcast(lo << 16, jnp.float32),
                plsc.bitcast(lo & jnp.int32(-65536), jnp.float32),
                plsc.bitcast(hi << 16, jnp.float32),
                plsc.bitcast(hi & jnp.int32(-65536), jnp.float32))

    def block_body(k, carry):
        row0 = k * _LANES
        scores = jnp.zeros((_LANES,), jnp.float32)
        quad0 = k * (_LANES // 4)
        for i in range(_LANES):
            rq = quad0 + i // 4
            col = (i % 4) * (EMBED_DIM // 2)
            hg = row_groups(hrows_v, rq, col)
            tg = row_groups(trows_v, rq, col)
            acc = hg[0] * tg[0] * rel_regs[0]
            for j in range(1, _DGROUPS):
                acc = acc + hg[j] * tg[j] * rel_regs[j]
            scores = scores + jnp.sum(acc) * onehot[i]
        out_v[pl.ds(row0, _LANES)] = scores
        return carry

    lax.fori_loop(0, _NBLOCKS, block_body, 0)

    pltpu.sync_copy(out_v, out_hbm.at[pl.ds(base, _ROWS_PER_W)])


_TBLK = 16384


def _tc_transpose_body(x_ref, o_ref):
    y32 = jax.lax.bitcast_convert_type(x_ref[...].T, jnp.int32)
    rne = (y32 + jnp.int32(0x7FFF) + ((y32 >> 16) & 1)) >> 16
    lo = rne[:, :EMBED_DIM // 2] & jnp.int32(0xFFFF)
    hi = rne[:, EMBED_DIM // 2:] << 16
    o_ref[...] = lo | hi


def _tc_transpose(table_t):
    grid = (N_NODES + _TBLK - 1) // _TBLK
    return pl.pallas_call(
        _tc_transpose_body,
        grid=(grid,),
        in_specs=[pl.BlockSpec((EMBED_DIM, _TBLK), lambda i: (0, i))],
        out_specs=pl.BlockSpec((_TBLK, EMBED_DIM // 2), lambda i: (i, 0)),
        out_shape=jax.ShapeDtypeStruct((N_NODES, EMBED_DIM // 2), jnp.int32),
    )(table_t)


@jax.jit
def _run(head_idx, tail_idx, table_t, rel_perm):
    table = _tc_transpose(table_t)
    mesh = plsc.VectorSubcoreMesh(core_axis_name="c", subcore_axis_name="s")
    kern = functools.partial(
        pl.kernel,
        mesh=mesh,
        compiler_params=pltpu.CompilerParams(needs_layout_passes=False),
        out_type=jax.ShapeDtypeStruct((BATCH,), jnp.float32),
        scratch_types=[
            pltpu.VMEM((_ROWS_PER_W,), jnp.int32),
            pltpu.VMEM((_ROWS_PER_W,), jnp.int32),
            pltpu.VMEM((_ROWS_PER_W // 4, 128), jnp.int32),
            pltpu.VMEM((_ROWS_PER_W // 4, 128), jnp.int32),
            pltpu.VMEM((EMBED_DIM,), jnp.float32),
            pltpu.VMEM((_ROWS_PER_W,), jnp.float32),
            pltpu.SemaphoreType.DMA,
        ],
    )(_sc_kernel)
    return kern(head_idx, tail_idx, table, rel_perm)


def kernel(head_indices, tail_indices, node_embedding, relation_vector):
    r = relation_vector
    rel_perm = jnp.concatenate([r[0:16], r[32:48], r[16:32], r[48:64]])
    return _run(head_indices.astype(jnp.int32),
                tail_indices.astype(jnp.int32),
                node_embedding.T, rel_perm)


# speedup vs baseline: 1.5215x; 1.5215x over previous
import functools

import jax
import jax.numpy as jnp
from jax import lax
from jax.experimental import pallas as pl
from jax.experimental.pallas import tpu as pltpu
from jax.experimental.pallas import tpu_sc as plsc

N_NODES = 1000000
EMBED_DIM = 64
BATCH = 16384

_INFO = plsc.get_sparse_core_info()
_NC = _INFO.num_cores
_NS = _INFO.num_subcores
_NW = _NC * _NS
_ROWS_PER_W = BATCH // _NW
_LANES = 16
_DGROUPS = EMBED_DIM // _LANES
_NBLOCKS = _ROWS_PER_W // _LANES


def _sc_kernel(head_hbm, tail_hbm, table_hbm, rel_hbm, out_hbm,
               hidx_v, tidx_v, hrows_v, trows_v, rel_v, out_v, sem):
    wid = lax.axis_index("s") * _NC + lax.axis_index("c")
    base = wid * _ROWS_PER_W

    pltpu.sync_copy(rel_hbm, rel_v)
    pltpu.sync_copy(head_hbm.at[pl.ds(base, _ROWS_PER_W)], hidx_v)
    pltpu.sync_copy(tail_hbm.at[pl.ds(base, _ROWS_PER_W)], tidx_v)

    def fire_group(g, carry):
        row0 = g * _LANES
        pair0 = g * (_LANES // 2)
        hv = hidx_v[pl.ds(row0, _LANES)]
        tv = tidx_v[pl.ds(row0, _LANES)]
        for i in range(_LANES):
            dst_row = pair0 + i // 2
            dst_col = (i % 2) * EMBED_DIM
            pltpu.async_copy(
                table_hbm.at[hv[i] >> 1],
                hrows_v.at[dst_row, pl.ds(dst_col, EMBED_DIM)], sem)
            pltpu.async_copy(
                table_hbm.at[tv[i] >> 1],
                trows_v.at[dst_row, pl.ds(dst_col, EMBED_DIM)], sem)
        return carry

    lax.fori_loop(0, _NBLOCKS, fire_group, 0)

    pltpu.make_async_copy(table_hbm.at[pl.ds(0, _ROWS_PER_W)], hrows_v,
                          sem).wait()
    pltpu.make_async_copy(table_hbm.at[pl.ds(0, _ROWS_PER_W)], trows_v,
                          sem).wait()

    rel_regs = [rel_v[pl.ds(j * _LANES, _LANES)] for j in range(_DGROUPS)]

    lane_iota = lax.iota(jnp.int32, _LANES)
    onehot = [(lane_iota == i).astype(jnp.float32) for i in range(_LANES)]

    def block_body(k, carry):
        pair0 = k * (_LANES // 2)
        scores = jnp.zeros((_LANES,), jnp.float32)
        row0 = k * _LANES
        hvv = hidx_v[pl.ds(row0, _LANES)]
        tvv = tidx_v[pl.ds(row0, _LANES)]
        for i in range(_LANES):
            brow = pair0 + i // 2
            bcol = (i % 2) * EMBED_DIM
            hs = (hvv[i] & 1) * 16
            ts = (tvv[i] & 1) * 16
            acc = None
            for j in range(_DGROUPS):
                hw = hrows_v[brow, pl.ds(bcol + j * _LANES, _LANES)]
                tw = trows_v[brow, pl.ds(bcol + j * _LANES, _LANES)]
                hf = plsc.bitcast((hw >> hs) << 16, jnp.float32)
                tf = plsc.bitcast((tw >> ts) << 16, jnp.float32)
                term = hf * tf * rel_regs[j]
                acc = term if acc is None else acc + term
            scores = scores + jnp.sum(acc) * onehot[i]
        out_v[pl.ds(k * _LANES, _LANES)] = scores
        return carry

    lax.fori_loop(0, _NBLOCKS, block_body, 0)

    pltpu.sync_copy(out_v, out_hbm.at[pl.ds(base, _ROWS_PER_W)])


_TBLK = 32768


def _tc_transpose_body(x_ref, o_ref):
    y = x_ref[...].T.astype(jnp.bfloat16)
    o_ref[...] = pltpu.bitcast(y, jnp.int32)


def _tc_transpose(table_t):
    grid = (N_NODES + _TBLK - 1) // _TBLK
    return pl.pallas_call(
        _tc_transpose_body,
        grid=(grid,),
        in_specs=[pl.BlockSpec((EMBED_DIM, _TBLK), lambda i: (0, i))],
        out_specs=pl.BlockSpec((_TBLK // 2, EMBED_DIM), lambda i: (i, 0)),
        out_shape=jax.ShapeDtypeStruct((N_NODES // 2, EMBED_DIM), jnp.int32),
    )(table_t)


@jax.jit
def _run(head_idx, tail_idx, table_t, rel):
    table = _tc_transpose(table_t)
    mesh = plsc.VectorSubcoreMesh(core_axis_name="c", subcore_axis_name="s")
    kern = functools.partial(
        pl.kernel,
        mesh=mesh,
        compiler_params=pltpu.CompilerParams(needs_layout_passes=False),
        out_type=jax.ShapeDtypeStruct((BATCH,), jnp.float32),
        scratch_types=[
            pltpu.VMEM((_ROWS_PER_W,), jnp.int32),
            pltpu.VMEM((_ROWS_PER_W,), jnp.int32),
            pltpu.VMEM((_ROWS_PER_W // 2, 2 * EMBED_DIM), jnp.int32),
            pltpu.VMEM((_ROWS_PER_W // 2, 2 * EMBED_DIM), jnp.int32),
            pltpu.VMEM((EMBED_DIM,), jnp.float32),
            pltpu.VMEM((_ROWS_PER_W,), jnp.float32),
            pltpu.SemaphoreType.DMA,
        ],
    )(_sc_kernel)
    return kern(head_idx, tail_idx, table, rel)


def kernel(head_indices, tail_indices, node_embedding, relation_vector):
    return _run(head_indices.astype(jnp.int32),
                tail_indices.astype(jnp.int32),
                node_embedding.T, relation_vector)
